# baseline (speedup 1.0000x reference)
import functools

import jax
import jax.numpy as jnp
from jax import lax
from jax.experimental import pallas as pl
from jax.experimental.pallas import tpu as pltpu
from jax.experimental.pallas import tpu_sc as plsc

N = 10000
NP = 10240
E = 320000
LANES = 16
NC = 2
NS = 16
F32 = jnp.float32


def _tc1_body(x_ref, w_ref, al_ref, ar_ref, feat_ref, proj_ref, *, H, D):
    xb = x_ref[...]
    fb = jnp.dot(xb, w_ref[...], preferred_element_type=F32)
    for h in range(H):
        fh = fb[:, h * D:(h + 1) * D]
        feat_ref[h] = fh
        proj_ref[h, :] = jnp.sum(fh * al_ref[h][None, :], axis=1)
        proj_ref[4 + h, :] = jnp.sum(fh * ar_ref[h][None, :], axis=1)


def _tc1(x, W0, al0p, ar0p):
    H, D = 4, 128
    bm = 1024
    grid = (NP // bm,)
    return pl.pallas_call(
        functools.partial(_tc1_body, H=H, D=D),
        grid=grid,
        in_specs=[
            pl.BlockSpec((bm, 128), lambda i: (i, 0)),
            pl.BlockSpec((128, H * D), lambda i: (0, 0)),
            pl.BlockSpec((8, D), lambda i: (0, 0)),
            pl.BlockSpec((8, D), lambda i: (0, 0)),
        ],
        out_specs=[
            pl.BlockSpec((H, bm, D), lambda i: (0, i, 0)),
            pl.BlockSpec((8, bm), lambda i: (0, i)),
        ],
        out_shape=[
            jax.ShapeDtypeStruct((H, NP, D), F32),
            jax.ShapeDtypeStruct((8, NP), F32),
        ],
    )(x, W0, al0p, ar0p)


def _tc2_body(rst_ref, b_ref, w_ref, al_ref, ar_ref, feat_ref, proj_ref):
    acc = jnp.zeros((rst_ref.shape[1], 128), F32)
    for h in range(4):
        hb = rst_ref[h] + b_ref[h][None, :]
        hb = jnp.where(hb > 0, hb, jnp.exp(hb) - 1.0)
        acc = acc + jnp.dot(hb, w_ref[h], preferred_element_type=F32)
    feat_ref[...] = acc
    proj_ref[0, :] = jnp.sum(acc * al_ref[0][None, :], axis=1)
    proj_ref[4, :] = jnp.sum(acc * ar_ref[0][None, :], axis=1)


def _tc2(rst0, b0r, W1r, al1p, ar1p):
    bm = 1024
    grid = (NP // bm,)
    return pl.pallas_call(
        _tc2_body,
        grid=grid,
        in_specs=[
            pl.BlockSpec((4, bm, 128), lambda i: (0, i, 0)),
            pl.BlockSpec((4, 128), lambda i: (0, 0)),
            pl.BlockSpec((4, 128, 128), lambda i: (0, 0, 0)),
            pl.BlockSpec((8, 128), lambda i: (0, 0)),
            pl.BlockSpec((8, 128), lambda i: (0, 0)),
        ],
        out_specs=[
            pl.BlockSpec((bm, 128), lambda i: (i, 0)),
            pl.BlockSpec((8, bm), lambda i: (0, i)),
        ],
        out_shape=[
            jax.ShapeDtypeStruct((NP, 128), F32),
            jax.ShapeDtypeStruct((8, NP), F32),
        ],
    )(rst0, b0r, W1r, al1p, ar1p)


def _tc3_body(rst_ref, b_ref, o_ref):
    y = rst_ref[0] + rst_ref[1] + b_ref[...][None, :]
    o_ref[...] = jnp.where(y > 0, y, jnp.exp(y) - 1.0)


def _tc3(rst1, b1):
    bm = 1000
    grid = (N // bm,)
    return pl.pallas_call(
        _tc3_body,
        grid=grid,
        in_specs=[
            pl.BlockSpec((2, bm, 128), lambda i: (0, i, 0)),
            pl.BlockSpec((128,), lambda i: (0,)),
        ],
        out_specs=pl.BlockSpec((bm, 128), lambda i: (i, 0)),
        out_shape=jax.ShapeDtypeStruct((N, 128), F32),
    )(rst1, b1)


_SC_PARAMS = pltpu.CompilerParams(needs_layout_passes=False)
_MESH = dict(core_axis_name="c", subcore_axis_name="s")


def _make_sc_denom(H, K=400):
    e_per_tile = E // (NC * NS)
    nchunks = e_per_tile // K
    ngroups = K // LANES
    mesh = plsc.VectorSubcoreMesh(**_MESH)

    @functools.partial(
        pl.kernel,
        out_type=[
            jax.ShapeDtypeStruct((NC * NS * H * NP,), F32),
            jax.ShapeDtypeStruct((H * E,), F32),
        ],
        mesh=mesh,
        compiler_params=_SC_PARAMS,
        scratch_types=[
            pltpu.VMEM((H * NP,), F32),
            pltpu.VMEM((H * NP,), F32),
            pltpu.VMEM((H * NP,), F32),
            pltpu.VMEM((K,), jnp.int32),
            pltpu.VMEM((K,), jnp.int32),
            pltpu.VMEM((H * K,), F32),
        ],
    )
    def sc_denom(src_hbm, dst_hbm, proj_hbm, den_hbm, ex_hbm,
                 el_v, er_v, den_v, sv, dv, exv):
        cc = lax.axis_index("c")
        ss = lax.axis_index("s")
        wid = cc * NS + ss
        ebase = wid * e_per_tile
        for h in range(H):
            pltpu.sync_copy(proj_hbm.at[pl.ds(h * NP, NP)],
                            el_v.at[pl.ds(h * NP, NP)])
            pltpu.sync_copy(proj_hbm.at[pl.ds((4 + h) * NP, NP)],
                            er_v.at[pl.ds(h * NP, NP)])

        def zbody(i, carry):
            den_v[pl.ds(i * LANES, LANES)] = jnp.zeros((LANES,), F32)
            return carry
        lax.fori_loop(0, H * NP // LANES, zbody, 0)

        def cbody(k, carry):
            pltpu.sync_copy(src_hbm.at[pl.ds(ebase + k * K, K)], sv)
            pltpu.sync_copy(dst_hbm.at[pl.ds(ebase + k * K, K)], dv)

            def gbody(g, c2):
                sl = pl.ds(g * LANES, LANES)
                s16 = sv[sl]
                d16 = dv[sl]
                for h in range(H):
                    sh = s16 + h * NP
                    dh = d16 + h * NP
                    e = (plsc.load_gather(el_v, [sh]) +
                         plsc.load_gather(er_v, [dh]))
                    e = jnp.where(e > 0, e, 0.2 * e)
                    ex = jnp.exp(e)
                    exv[pl.ds(h * K + g * LANES, LANES)] = ex
                    plsc.addupdate_scatter(den_v, [dh], ex)
                return c2
            lax.fori_loop(0, ngroups, gbody, 0)
            for h in range(H):
                pltpu.sync_copy(
                    exv.at[pl.ds(h * K, K)],
                    ex_hbm.at[pl.ds(h * E + ebase + k * K, K)])
            return carry
        lax.fori_loop(0, nchunks, cbody, 0)
        for h in range(H):
            pltpu.sync_copy(den_v.at[pl.ds(h * NP, NP)],
                            den_hbm.at[pl.ds((wid * H + h) * NP, NP)])

    return sc_denom


def _make_sc_reduce(H):
    total = H * NP
    per_tile = total // (NC * NS)
    mesh = plsc.VectorSubcoreMesh(**_MESH)

    @functools.partial(
        pl.kernel,
        out_type=jax.ShapeDtypeStruct((total,), F32),
        mesh=mesh,
        compiler_params=_SC_PARAMS,
        scratch_types=[
            pltpu.VMEM((per_tile,), F32),
            pltpu.VMEM((per_tile,), F32),
        ],
    )
    def sc_reduce(den_hbm, inv_hbm, acc_v, tmp_v):
        cc = lax.axis_index("c")
        ss = lax.axis_index("s")
        wid = cc * NS + ss
        base = wid * per_tile

        def zbody(i, carry):
            acc_v[pl.ds(i * LANES, LANES)] = jnp.zeros((LANES,), F32)
            return carry
        lax.fori_loop(0, per_tile // LANES, zbody, 0)

        def tbody(t, carry):
            pltpu.sync_copy(den_hbm.at[pl.ds(t * total + base, per_tile)],
                            tmp_v)

            def vbody(i, c2):
                sl = pl.ds(i * LANES, LANES)
                acc_v[sl] = acc_v[sl] + tmp_v[sl]
                return c2
            lax.fori_loop(0, per_tile // LANES, vbody, 0)
            return carry
        lax.fori_loop(0, NC * NS, tbody, 0)

        def ibody(i, carry):
            sl = pl.ds(i * LANES, LANES)
            acc_v[sl] = 1.0 / (acc_v[sl] + 1e-9)
            return carry
        lax.fori_loop(0, per_tile // LANES, ibody, 0)
        pltpu.sync_copy(acc_v, inv_hbm.at[pl.ds(base, per_tile)])

    return sc_reduce


def _make_sc_agg(n_tab, D, P, edge_split=False, K=80, BC=10, R=3):
    e_per_tile = E // (NC * NS) if edge_split else E // NS
    BK = BC * K
    nblocks = e_per_tile // BK
    tailc = (e_per_tile - nblocks * BK) // K
    ngroups = K // LANES
    DG = D // LANES
    rows_per_tile = NP // NS
    zrows = 16
    mesh = plsc.VectorSubcoreMesh(**_MESH)

    @functools.partial(
        pl.kernel,
        out_type=jax.ShapeDtypeStruct(
            ((NC if edge_split else n_tab) * NP, D), F32),
        mesh=mesh,
        compiler_params=_SC_PARAMS,
        scratch_types=[
            pltpu.VMEM((NP,), F32),
            pltpu.VMEM((BK,), jnp.int32),
            pltpu.VMEM((BC, K), jnp.int32),
            pltpu.VMEM((BC, K), jnp.int32),
            pltpu.VMEM((BK,), F32),
            pltpu.VMEM((R * K, D), F32),
            pltpu.VMEM((zrows, D), F32),
            pltpu.VMEM_SHARED((NP, D), F32),
            pltpu.SemaphoreType.DMA,
            pltpu.SemaphoreType.DMA,
        ],
    )
    def sc_agg(src_hbm, dst_hbm, ex_hbm, inv_hbm, feat_hbm, rst_hbm,
               inv_v, idxb, sivb, dvb, avf, rows_v, zb, acc, gsem, ssem):
        cc = lax.axis_index("c")
        ss = lax.axis_index("s")
        ebase = ((cc * NS + ss) if edge_split else ss) * e_per_tile

        for i in range(zrows):
            for f in range(DG):
                zb[i, pl.ds(f * LANES, LANES)] = jnp.zeros((LANES,), F32)

        def slot(j):
            return j - (j // R) * R

        def gather_cp(j, r):
            return pltpu.make_async_copy(
                feat_hbm.at[sivb.at[j]],
                rows_v.at[pl.ds(r * K, K)], gsem)

        def scatter_cp(j, r):
            return pltpu.make_async_copy(
                rows_v.at[pl.ds(r * K, K)], acc.at[dvb.at[j]], ssem)

        def do_block(base, tab, abase, nch):
            pltpu.sync_copy(src_hbm.at[pl.ds(base, nch * K)],
                            idxb.at[pl.ds(0, nch * K)])
            pltpu.sync_copy(ex_hbm.at[pl.ds(abase + base, nch * K)],
                            avf.at[pl.ds(0, nch * K)])

            def ibody(g, c2):
                u = g // ngroups
                gg = g - u * ngroups
                sivb[u, pl.ds(gg * LANES, LANES)] = (
                    idxb[pl.ds(g * LANES, LANES)] + tab * NP)
                return c2
            lax.fori_loop(0, nch * ngroups, ibody, 0)
            pltpu.async_copy(feat_hbm.at[sivb.at[0]],
                             rows_v.at[pl.ds(0, K)], gsem)

            pltpu.sync_copy(dst_hbm.at[pl.ds(base, nch * K)],
                            idxb.at[pl.ds(0, nch * K)])

            def i2body(g, c2):
                u = g // ngroups
                gg = g - u * ngroups
                sl16 = pl.ds(g * LANES, LANES)
                d16 = idxb[sl16]
                dvb[u, pl.ds(gg * LANES, LANES)] = d16
                avf[sl16] = avf[sl16] * plsc.load_gather(inv_v, [d16])
                return c2
            lax.fori_loop(0, nch * ngroups, i2body, 0)

            def jbody(j, c2):
                r = slot(j)

                @pl.when(j >= 2)
                def _():
                    scatter_cp(j - 2, slot(j + 1)).wait()

                @pl.when(j + 1 < nch)
                def _():
                    pltpu.async_copy(
                        feat_hbm.at[sivb.at[j + 1]],
                        rows_v.at[pl.ds(slot(j + 1) * K, K)], gsem)

                gather_cp(j, r).wait()

                def sbody(g, c3):
                    a16 = avf[pl.ds(j * K + g * LANES, LANES)]
                    for l in range(LANES):
                        a = a16[l]
                        row = r * K + g * LANES + l
                        for f in range(DG):
                            sl = pl.ds(f * LANES, LANES)
                            rows_v[row, sl] = rows_v[row, sl] * a
                    return c3
                lax.fori_loop(0, ngroups, sbody, 0)
                pltpu.async_copy(
                    rows_v.at[pl.ds(r * K, K)], acc.at[dvb.at[j]],
                    ssem, add=True)
                return c2
            lax.fori_loop(0, nch, jbody, 0)
            if nch >= 2:
                scatter_cp(nch - 2, slot(nch - 2)).wait()
            scatter_cp(nch - 1, slot(nch - 1)).wait()

        for p in range(P):
            tab = (cc * P + p) * (0 if edge_split else 1)
            outb = cc if edge_split else tab
            abase = 0 if edge_split else tab * E
            pltpu.sync_copy(inv_hbm.at[pl.ds(tab * NP, NP)], inv_v)

            zslice = NP // NS

            def zc(i, carry):
                pltpu.sync_copy(
                    zb, acc.at[pl.ds(ss * zslice + i * zrows, zrows)])
                return carry
            lax.fori_loop(0, zslice // zrows, zc, 0)
            plsc.subcore_barrier()

            def cbody(m, carry):
                do_block(ebase + m * BK, tab, abase, BC)
                return carry
            lax.fori_loop(0, nblocks, cbody, 0)
            if tailc:
                do_block(ebase + nblocks * BK, tab, abase, tailc)
            plsc.subcore_barrier()

            rbase = ss * rows_per_tile
            pltpu.sync_copy(
                acc.at[pl.ds(rbase, rows_per_tile)],
                rst_hbm.at[pl.ds(outb * NP + rbase, rows_per_tile)])
            plsc.subcore_barrier()

    return sc_agg


_sc_denom0 = _make_sc_denom(H=4)
_sc_denom1 = _make_sc_denom(H=1)
_sc_reduce0 = _make_sc_reduce(H=4)
_sc_reduce1 = _make_sc_reduce(H=1)
_sc_agg0 = _make_sc_agg(n_tab=4, D=128, P=2)
_sc_agg1 = _make_sc_agg(n_tab=1, D=128, P=1, edge_split=True)


def _pad_rows(a):
    out = jnp.zeros((8, a.shape[1]), F32)
    return out.at[:a.shape[0]].set(a)


def kernel(x, edge_index_0, edge_index_1, W0, al0, ar0, b0, W1, al1, ar1, b1):
    src0, dst0 = edge_index_0[0], edge_index_0[1]
    src1, dst1 = edge_index_1[0], edge_index_1[1]

    feat0, proj0 = _tc1(x, W0, _pad_rows(al0), _pad_rows(ar0))
    proj0f = proj0.reshape(8 * NP)
    den0, ex0 = _sc_denom0(src0, dst0, proj0f)
    inv0 = _sc_reduce0(den0)
    rst0 = _sc_agg0(src0, dst0, ex0, inv0, feat0.reshape(4 * NP, 128))

    feat1, proj1 = _tc2(rst0.reshape(4, NP, 128), b0.reshape(4, 128),
                        W1.reshape(4, 128, 128), _pad_rows(al1),
                        _pad_rows(ar1))
    proj1f = proj1.reshape(8 * NP)
    den1, ex1 = _sc_denom1(src1, dst1, proj1f)
    inv1 = _sc_reduce1(den1)
    rst1 = _sc_agg1(src1, dst1, ex1, inv1, feat1)

    return _tc3(rst1.reshape(2, NP, 128), b1)


# speedup vs baseline: 1.0166x; 1.0166x over previous
import functools

import jax
import jax.numpy as jnp
from jax import lax
from jax.experimental import pallas as pl
from jax.experimental.pallas import tpu as pltpu
from jax.experimental.pallas import tpu_sc as plsc

N = 10000
NP = 10240
E = 320000
LANES = 16
NC = 2
NS = 16
F32 = jnp.float32


def _tc1_body(x_ref, w_ref, al_ref, ar_ref, feat_ref, proj_ref, *, H, D):
    xb = x_ref[...]
    fb = jnp.dot(xb, w_ref[...], preferred_element_type=F32)
    for h in range(H):
        fh = fb[:, h * D:(h + 1) * D]
        feat_ref[h] = fh
        proj_ref[h, :] = jnp.sum(fh * al_ref[h][None, :], axis=1)
        proj_ref[4 + h, :] = jnp.sum(fh * ar_ref[h][None, :], axis=1)


def _tc1(x, W0, al0p, ar0p):
    H, D = 4, 128
    bm = 1024
    grid = (NP // bm,)
    return pl.pallas_call(
        functools.partial(_tc1_body, H=H, D=D),
        grid=grid,
        in_specs=[
            pl.BlockSpec((bm, 128), lambda i: (i, 0)),
            pl.BlockSpec((128, H * D), lambda i: (0, 0)),
            pl.BlockSpec((8, D), lambda i: (0, 0)),
            pl.BlockSpec((8, D), lambda i: (0, 0)),
        ],
        out_specs=[
            pl.BlockSpec((H, bm, D), lambda i: (0, i, 0)),
            pl.BlockSpec((8, bm), lambda i: (0, i)),
        ],
        out_shape=[
            jax.ShapeDtypeStruct((H, NP, D), F32),
            jax.ShapeDtypeStruct((8, NP), F32),
        ],
    )(x, W0, al0p, ar0p)


def _tc2_body(rst_ref, b_ref, w_ref, al_ref, ar_ref, feat_ref, proj_ref):
    acc = jnp.zeros((rst_ref.shape[1], 128), F32)
    for h in range(4):
        hb = rst_ref[h] + b_ref[h][None, :]
        hb = jnp.where(hb > 0, hb, jnp.exp(hb) - 1.0)
        acc = acc + jnp.dot(hb, w_ref[h], preferred_element_type=F32)
    feat_ref[...] = acc
    proj_ref[0, :] = jnp.sum(acc * al_ref[0][None, :], axis=1)
    proj_ref[4, :] = jnp.sum(acc * ar_ref[0][None, :], axis=1)


def _tc2(rst0, b0r, W1r, al1p, ar1p):
    bm = 1024
    grid = (NP // bm,)
    return pl.pallas_call(
        _tc2_body,
        grid=grid,
        in_specs=[
            pl.BlockSpec((4, bm, 128), lambda i: (0, i, 0)),
            pl.BlockSpec((4, 128), lambda i: (0, 0)),
            pl.BlockSpec((4, 128, 128), lambda i: (0, 0, 0)),
            pl.BlockSpec((8, 128), lambda i: (0, 0)),
            pl.BlockSpec((8, 128), lambda i: (0, 0)),
        ],
        out_specs=[
            pl.BlockSpec((bm, 128), lambda i: (i, 0)),
            pl.BlockSpec((8, bm), lambda i: (0, i)),
        ],
        out_shape=[
            jax.ShapeDtypeStruct((NP, 128), F32),
            jax.ShapeDtypeStruct((8, NP), F32),
        ],
    )(rst0, b0r, W1r, al1p, ar1p)


def _tc3_body(rst_ref, b_ref, o_ref):
    y = rst_ref[0] + rst_ref[1] + b_ref[...][None, :]
    o_ref[...] = jnp.where(y > 0, y, jnp.exp(y) - 1.0)


def _tc3(rst1, b1):
    bm = 1000
    grid = (N // bm,)
    return pl.pallas_call(
        _tc3_body,
        grid=grid,
        in_specs=[
            pl.BlockSpec((2, bm, 128), lambda i: (0, i, 0)),
            pl.BlockSpec((128,), lambda i: (0,)),
        ],
        out_specs=pl.BlockSpec((bm, 128), lambda i: (i, 0)),
        out_shape=jax.ShapeDtypeStruct((N, 128), F32),
    )(rst1, b1)


_SC_PARAMS = pltpu.CompilerParams(needs_layout_passes=False)
_MESH = dict(core_axis_name="c", subcore_axis_name="s")


def _make_sc_denom(H, K=2000):
    e_per_tile = E // (NC * NS)
    nchunks = e_per_tile // K
    ngroups = K // LANES
    mesh = plsc.VectorSubcoreMesh(**_MESH)

    @functools.partial(
        pl.kernel,
        out_type=[
            jax.ShapeDtypeStruct((NC * NS * H * NP,), F32),
            jax.ShapeDtypeStruct((H * E,), F32),
        ],
        mesh=mesh,
        compiler_params=_SC_PARAMS,
        scratch_types=[
            pltpu.VMEM((NP,), F32),
            pltpu.VMEM((NP,), F32),
            pltpu.VMEM((NP,), F32),
            pltpu.VMEM((K,), jnp.int32),
            pltpu.VMEM((K,), jnp.int32),
            pltpu.VMEM((K,), F32),
        ],
    )
    def sc_denom(src_hbm, dst_hbm, proj_hbm, den_hbm, ex_hbm,
                 el_v, er_v, den_v, sv, dv, exv):
        cc = lax.axis_index("c")
        ss = lax.axis_index("s")
        wid = cc * NS + ss
        ebase = wid * e_per_tile
        for h in range(H):
            pltpu.sync_copy(proj_hbm.at[pl.ds(h * NP, NP)], el_v)
            pltpu.sync_copy(proj_hbm.at[pl.ds((4 + h) * NP, NP)], er_v)

            def zbody(i, carry):
                den_v[pl.ds(i * LANES, LANES)] = jnp.zeros((LANES,), F32)
                return carry
            lax.fori_loop(0, NP // LANES, zbody, 0)

            def cbody(k, carry):
                pltpu.sync_copy(src_hbm.at[pl.ds(ebase + k * K, K)], sv)
                pltpu.sync_copy(dst_hbm.at[pl.ds(ebase + k * K, K)], dv)

                def gbody(g, c2):
                    sl = pl.ds(g * LANES, LANES)
                    s16 = sv[sl]
                    d16 = dv[sl]
                    e = (plsc.load_gather(el_v, [s16]) +
                         plsc.load_gather(er_v, [d16]))
                    e = jnp.where(e > 0, e, 0.2 * e)
                    ex = jnp.exp(e)
                    exv[sl] = ex
                    plsc.addupdate_scatter(den_v, [d16], ex)
                    return c2
                lax.fori_loop(0, ngroups, gbody, 0)
                pltpu.sync_copy(
                    exv, ex_hbm.at[pl.ds(h * E + ebase + k * K, K)])
                return carry
            lax.fori_loop(0, nchunks, cbody, 0)
            pltpu.sync_copy(den_v, den_hbm.at[pl.ds((wid * H + h) * NP, NP)])

    return sc_denom


def _make_sc_reduce(H):
    total = H * NP
    per_tile = total // (NC * NS)
    mesh = plsc.VectorSubcoreMesh(**_MESH)

    @functools.partial(
        pl.kernel,
        out_type=jax.ShapeDtypeStruct((total,), F32),
        mesh=mesh,
        compiler_params=_SC_PARAMS,
        scratch_types=[
            pltpu.VMEM((per_tile,), F32),
            pltpu.VMEM((per_tile,), F32),
        ],
    )
    def sc_reduce(den_hbm, inv_hbm, acc_v, tmp_v):
        cc = lax.axis_index("c")
        ss = lax.axis_index("s")
        wid = cc * NS + ss
        base = wid * per_tile

        def zbody(i, carry):
            acc_v[pl.ds(i * LANES, LANES)] = jnp.zeros((LANES,), F32)
            return carry
        lax.fori_loop(0, per_tile // LANES, zbody, 0)

        def tbody(t, carry):
            pltpu.sync_copy(den_hbm.at[pl.ds(t * total + base, per_tile)],
                            tmp_v)

            def vbody(i, c2):
                sl = pl.ds(i * LANES, LANES)
                acc_v[sl] = acc_v[sl] + tmp_v[sl]
                return c2
            lax.fori_loop(0, per_tile // LANES, vbody, 0)
            return carry
        lax.fori_loop(0, NC * NS, tbody, 0)

        def ibody(i, carry):
            sl = pl.ds(i * LANES, LANES)
            acc_v[sl] = 1.0 / (acc_v[sl] + 1e-9)
            return carry
        lax.fori_loop(0, per_tile // LANES, ibody, 0)
        pltpu.sync_copy(acc_v, inv_hbm.at[pl.ds(base, per_tile)])

    return sc_reduce


def _make_sc_agg(n_tab, D, P, edge_split=False, K=80, BC=10, R=3):
    e_per_tile = E // (NC * NS) if edge_split else E // NS
    BK = BC * K
    nblocks = e_per_tile // BK
    tailc = (e_per_tile - nblocks * BK) // K
    ngroups = K // LANES
    DG = D // LANES
    rows_per_tile = NP // NS
    zrows = 16
    mesh = plsc.VectorSubcoreMesh(**_MESH)

    @functools.partial(
        pl.kernel,
        out_type=jax.ShapeDtypeStruct(
            ((NC if edge_split else n_tab) * NP, D), F32),
        mesh=mesh,
        compiler_params=_SC_PARAMS,
        scratch_types=[
            pltpu.VMEM((NP,), F32),
            pltpu.VMEM((BK,), jnp.int32),
            pltpu.VMEM((BC, K), jnp.int32),
            pltpu.VMEM((BC, K), jnp.int32),
            pltpu.VMEM((BK,), F32),
            pltpu.VMEM((R * K, D), F32),
            pltpu.VMEM((zrows, D), F32),
            pltpu.VMEM_SHARED((NP, D), F32),
            pltpu.SemaphoreType.DMA,
            pltpu.SemaphoreType.DMA,
        ],
    )
    def sc_agg(src_hbm, dst_hbm, ex_hbm, inv_hbm, feat_hbm, rst_hbm,
               inv_v, idxb, sivb, dvb, avf, rows_v, zb, acc, gsem, ssem):
        cc = lax.axis_index("c")
        ss = lax.axis_index("s")
        ebase = ((cc * NS + ss) if edge_split else ss) * e_per_tile

        for i in range(zrows):
            for f in range(DG):
                zb[i, pl.ds(f * LANES, LANES)] = jnp.zeros((LANES,), F32)

        def slot(j):
            return j - (j // R) * R

        def gather_cp(j, r):
            return pltpu.make_async_copy(
                feat_hbm.at[sivb.at[j]],
                rows_v.at[pl.ds(r * K, K)], gsem)

        def scatter_cp(j, r):
            return pltpu.make_async_copy(
                rows_v.at[pl.ds(r * K, K)], acc.at[dvb.at[j]], ssem)

        def do_block(base, tab, abase, nch):
            pltpu.sync_copy(src_hbm.at[pl.ds(base, nch * K)],
                            idxb.at[pl.ds(0, nch * K)])
            pltpu.sync_copy(ex_hbm.at[pl.ds(abase + base, nch * K)],
                            avf.at[pl.ds(0, nch * K)])

            def ibody(g, c2):
                u = g // ngroups
                gg = g - u * ngroups
                sivb[u, pl.ds(gg * LANES, LANES)] = (
                    idxb[pl.ds(g * LANES, LANES)] + tab * NP)
                return c2
            lax.fori_loop(0, nch * ngroups, ibody, 0)
            pltpu.async_copy(feat_hbm.at[sivb.at[0]],
                             rows_v.at[pl.ds(0, K)], gsem)

            pltpu.sync_copy(dst_hbm.at[pl.ds(base, nch * K)],
                            idxb.at[pl.ds(0, nch * K)])

            def i2body(g, c2):
                u = g // ngroups
                gg = g - u * ngroups
                sl16 = pl.ds(g * LANES, LANES)
                d16 = idxb[sl16]
                dvb[u, pl.ds(gg * LANES, LANES)] = d16
                avf[sl16] = avf[sl16] * plsc.load_gather(inv_v, [d16])
                return c2
            lax.fori_loop(0, nch * ngroups, i2body, 0)

            def jbody(j, c2):
                r = slot(j)

                @pl.when(j >= 2)
                def _():
                    scatter_cp(j - 2, slot(j + 1)).wait()

                @pl.when(j + 1 < nch)
                def _():
                    pltpu.async_copy(
                        feat_hbm.at[sivb.at[j + 1]],
                        rows_v.at[pl.ds(slot(j + 1) * K, K)], gsem)

                gather_cp(j, r).wait()

                def sbody(g, c3):
                    a16 = avf[pl.ds(j * K + g * LANES, LANES)]
                    for l in range(LANES):
                        a = a16[l]
                        row = r * K + g * LANES + l
                        for f in range(DG):
                            sl = pl.ds(f * LANES, LANES)
                            rows_v[row, sl] = rows_v[row, sl] * a
                    return c3
                lax.fori_loop(0, ngroups, sbody, 0)
                pltpu.async_copy(
                    rows_v.at[pl.ds(r * K, K)], acc.at[dvb.at[j]],
                    ssem, add=True)
                return c2
            lax.fori_loop(0, nch, jbody, 0)
            if nch >= 2:
                scatter_cp(nch - 2, slot(nch - 2)).wait()
            scatter_cp(nch - 1, slot(nch - 1)).wait()

        for p in range(P):
            tab = (cc * P + p) * (0 if edge_split else 1)
            outb = cc if edge_split else tab
            abase = 0 if edge_split else tab * E
            pltpu.sync_copy(inv_hbm.at[pl.ds(tab * NP, NP)], inv_v)

            zslice = NP // NS

            def zc(i, carry):
                pltpu.sync_copy(
                    zb, acc.at[pl.ds(ss * zslice + i * zrows, zrows)])
                return carry
            lax.fori_loop(0, zslice // zrows, zc, 0)
            plsc.subcore_barrier()

            def cbody(m, carry):
                do_block(ebase + m * BK, tab, abase, BC)
                return carry
            lax.fori_loop(0, nblocks, cbody, 0)
            if tailc:
                do_block(ebase + nblocks * BK, tab, abase, tailc)
            plsc.subcore_barrier()

            rbase = ss * rows_per_tile
            pltpu.sync_copy(
                acc.at[pl.ds(rbase, rows_per_tile)],
                rst_hbm.at[pl.ds(outb * NP + rbase, rows_per_tile)])
            plsc.subcore_barrier()

    return sc_agg


_sc_denom0 = _make_sc_denom(H=4)
_sc_denom1 = _make_sc_denom(H=1)
_sc_reduce0 = _make_sc_reduce(H=4)
_sc_reduce1 = _make_sc_reduce(H=1)
_sc_agg0 = _make_sc_agg(n_tab=4, D=128, P=2)
_sc_agg1 = _make_sc_agg(n_tab=1, D=128, P=1, edge_split=True)


def _pad_rows(a):
    out = jnp.zeros((8, a.shape[1]), F32)
    return out.at[:a.shape[0]].set(a)


def kernel(x, edge_index_0, edge_index_1, W0, al0, ar0, b0, W1, al1, ar1, b1):
    src0, dst0 = edge_index_0[0], edge_index_0[1]
    src1, dst1 = edge_index_1[0], edge_index_1[1]

    feat0, proj0 = _tc1(x, W0, _pad_rows(al0), _pad_rows(ar0))
    proj0f = proj0.reshape(8 * NP)
    den0, ex0 = _sc_denom0(src0, dst0, proj0f)
    inv0 = _sc_reduce0(den0)
    rst0 = _sc_agg0(src0, dst0, ex0, inv0, feat0.reshape(4 * NP, 128))

    feat1, proj1 = _tc2(rst0.reshape(4, NP, 128), b0.reshape(4, 128),
                        W1.reshape(4, 128, 128), _pad_rows(al1),
                        _pad_rows(ar1))
    proj1f = proj1.reshape(8 * NP)
    den1, ex1 = _sc_denom1(src1, dst1, proj1f)
    inv1 = _sc_reduce1(den1)
    rst1 = _sc_agg1(src1, dst1, ex1, inv1, feat1)

    return _tc3(rst1.reshape(2, NP, 128), b1)
